# Initial kernel scaffold; baseline (speedup 1.0000x reference)
import jax
import jax.numpy as jnp
from jax.experimental import pallas as pl


def kernel(text, offsets, table, W, b):
    raise NotImplementedError("write your pallas kernel here")


# speedup vs baseline: 1.7142x; 1.7142x over previous
import functools

import jax
import jax.numpy as jnp
from jax import lax
from jax.experimental import pallas as pl
from jax.experimental.pallas import tpu as pltpu
from jax.experimental.pallas import tpu_sc as plsc

_LANES = 16


@functools.partial(jax.jit, static_argnames=("chunk", "chunks_per_bag"))
def _sc_bag_sums(idx3, table, *, chunk, chunks_per_bag):
    nw, chunks_per_w, _ = idx3.shape
    d = table.shape[1]
    bags_per_w = chunks_per_w // chunks_per_bag
    d_vregs = d // _LANES
    mesh = plsc.VectorSubcoreMesh(core_axis_name="c", subcore_axis_name="s")
    nc = mesh.num_cores

    @functools.partial(
        pl.kernel,
        out_type=jax.ShapeDtypeStruct((nw, bags_per_w, d), jnp.float32),
        mesh=mesh,
        scratch_types=[
            pltpu.VMEM((chunks_per_w, chunk), jnp.int32),
            pltpu.VMEM((chunk, d), jnp.float32),
            pltpu.VMEM((bags_per_w, d), jnp.float32),
            pltpu.SemaphoreType.DMA,
        ],
        compiler_params=pltpu.CompilerParams(use_tc_tiling_on_sc=False),
    )
    def body(idx_hbm, table_hbm, out_hbm, idx_v, rows_v, out_v, sem):
        wid = lax.axis_index("s") * nc + lax.axis_index("c")
        pltpu.sync_copy(idx_hbm.at[wid], idx_v)

        def bag_body(bag, carry):
            def chunk_body(h, accs):
                ch = bag * chunks_per_bag + h
                pltpu.async_copy(table_hbm.at[idx_v.at[ch]], rows_v, sem).wait()

                def row_body(i, a):
                    return tuple(
                        a[v] + rows_v[i, pl.ds(v * _LANES, _LANES)]
                        for v in range(d_vregs)
                    )

                return lax.fori_loop(0, chunk, row_body, accs)

            zeros = (jnp.zeros((_LANES,), jnp.float32),) * d_vregs
            accs = lax.fori_loop(0, chunks_per_bag, chunk_body, zeros)
            for v in range(d_vregs):
                out_v[bag, pl.ds(v * _LANES, _LANES)] = accs[v]
            return carry

        lax.fori_loop(0, bags_per_w, bag_body, 0)
        pltpu.sync_copy(out_v, out_hbm.at[wid])

    return body(idx3, table)


def _proj_body(x_ref, w_ref, b_ref, o_ref, *, scale):
    acc = lax.dot_general(
        x_ref[...], w_ref[...], (((1,), (1,)), ((), ())),
        preferred_element_type=jnp.float32,
    )
    o_ref[...] = acc * scale + b_ref[...]


@functools.partial(jax.jit, static_argnames=("scale",))
def _tc_project(x, w, b2, *, scale):
    bdim, _ = x.shape
    c = w.shape[0]
    return pl.pallas_call(
        functools.partial(_proj_body, scale=scale),
        out_shape=jax.ShapeDtypeStruct((bdim, c), jnp.float32),
    )(x, w, b2)


def kernel(text, offsets, table, W, b):
    bdim, seq = text.shape
    d = table.shape[1]
    nw = 32
    chunk = 100
    chunks_per_bag = seq // chunk
    flat = text.reshape(-1).astype(jnp.int32)
    idx3 = flat.reshape(nw, (bdim * seq) // (nw * chunk), chunk)
    sums = _sc_bag_sums(idx3, table, chunk=chunk, chunks_per_bag=chunks_per_bag)
    x = sums.reshape(bdim, d)
    return _tc_project(x, W, b.reshape(1, -1).astype(jnp.float32),
                       scale=1.0 / seq)
